# baseline (speedup 1.0000x reference)
import functools
import jax
import jax.numpy as jnp
from jax import lax
from jax.experimental import pallas as pl
from jax.experimental.pallas import tpu as pltpu
from jax.experimental.pallas import tpu_sc as plsc

_B, _N, _L = 16, 1024, 512
_NS = 4
_C = _L // _NS
_NB = 512
_NW = 32
_RW = _N // _NW


def _perm_matrix(L, ns):
    c = L // ns
    src = jnp.arange(L)
    dst = (src % ns) * c + src // ns
    return jnp.zeros((L, L), jnp.float32).at[src, dst].set(1.0)


def _pre_body(e_ref, p_ref, ed_ref):
    ed_ref[...] = jnp.dot(e_ref[...], p_ref[...],
                          preferred_element_type=jnp.float32)


def _tc_body(x_ref, ed_ref, p_ref, o1_ref):
    xb = x_ref[0]
    ed = ed_ref[...]
    pm = p_ref[...]
    xd = jnp.dot(xb, pm, preferred_element_type=jnp.float32)
    rows1 = [jnp.concatenate([xd[:, s * _C:(s + 1) * _C],
                              ed[:, s * _C:(s + 1) * _C]], axis=1)
             for s in range(_NS)]
    o1_ref[0] = jnp.stack(rows1, axis=1)


def _tc_out1(x, node_emb):
    B, N, L = x.shape
    pm = _perm_matrix(L, _NS)
    ed_all = pl.pallas_call(
        _pre_body,
        grid=(N // _NB,),
        in_specs=[
            pl.BlockSpec((_NB, L), lambda i: (i, 0)),
            pl.BlockSpec((L, L), lambda i: (0, 0)),
        ],
        out_specs=pl.BlockSpec((_NB, L), lambda i: (i, 0)),
        out_shape=jax.ShapeDtypeStruct((N, L), jnp.float32),
    )(node_emb, pm)

    out4 = jax.ShapeDtypeStruct((B, N, _NS, 2 * _C), x.dtype)
    return pl.pallas_call(
        _tc_body,
        grid=(N // _NB, B),
        in_specs=[
            pl.BlockSpec((1, _NB, L), lambda i, b: (b, i, 0)),
            pl.BlockSpec((_NB, L), lambda i, b: (i, 0)),
            pl.BlockSpec((L, L), lambda i, b: (0, 0)),
        ],
        out_specs=pl.BlockSpec((1, _NB, _NS, 2 * _C),
                               lambda i, b: (b, i, 0, 0)),
        out_shape=out4,
    )(x, ed_all, pm)


def _sc_out2(x, node_emb):
    mesh = plsc.VectorSubcoreMesh(core_axis_name="c", subcore_axis_name="s")
    out_sd = jax.ShapeDtypeStruct((_B, _N, _NS, 2 * _C), jnp.float32)

    @functools.partial(
        pl.kernel, mesh=mesh,
        out_type=out_sd,
        scratch_types=[
            pltpu.VMEM((2, _RW, _L), jnp.float32),
            pltpu.VMEM((2, _RW, _NS, 2 * _C), jnp.float32),
            pltpu.SemaphoreType.DMA((2,)),
            pltpu.SemaphoreType.DMA((2,)),
        ],
    )
    def k(x_hbm, e_hbm, o2_hbm, xv, s2, sin, sout):
        wid = lax.axis_index("s") * 2 + lax.axis_index("c")
        n0 = wid * _RW

        def in_copy(b, buf):
            return pltpu.make_async_copy(
                x_hbm.at[b, pl.ds(n0, _RW)], xv.at[buf], sin.at[buf])

        def out_copy(b, buf):
            return pltpu.make_async_copy(
                s2.at[buf], o2_hbm.at[b, pl.ds(n0, _RW)], sout.at[buf])

        def fill(buf, r, xoff):
            for s in range(_NS):
                for g in range(_C // 16):
                    v = xv[buf, r, pl.ds(s * _C + g * 16, 16)]
                    s2[buf, r, s, pl.ds(xoff + g * 16, 16)] = v

        pltpu.sync_copy(e_hbm.at[pl.ds(n0, _RW)], xv.at[0])

        def fill_emb(r, carry):
            fill(0, r, _C)
            return carry

        lax.fori_loop(0, _RW, fill_emb, 0)

        def fill_emb1(r, carry):
            for s in range(_NS):
                for g in range(_C // 16):
                    v = s2[0, r, s, pl.ds(_C + g * 16, 16)]
                    s2[1, r, s, pl.ds(_C + g * 16, 16)] = v
            return carry

        lax.fori_loop(0, _RW, fill_emb1, 0)

        in_copy(0, 0).start()
        for b in range(_B):
            buf = b % 2
            in_copy(b, buf).wait()
            if b + 1 < _B:
                in_copy(b + 1, 1 - buf).start()
            if b >= 2:
                out_copy(b - 2, buf).wait()

            def fill_x(r, c2, buf=buf):
                fill(buf, r, 0)
                return c2

            lax.fori_loop(0, _RW, fill_x, 0)
            out_copy(b, buf).start()
        out_copy(_B - 2, 0).wait()
        out_copy(_B - 1, 1).wait()

    return k(x, node_emb)


def kernel(x, x_mark, node_emb):
    del x_mark
    o1 = _tc_out1(x, node_emb)
    return o1, o1


# speedup vs baseline: 1.1966x; 1.1966x over previous
import functools
import jax
import jax.numpy as jnp
from jax import lax
from jax.experimental import pallas as pl
from jax.experimental.pallas import tpu as pltpu
from jax.experimental.pallas import tpu_sc as plsc

_B, _N, _L = 16, 1024, 512
_NS = 4
_C = _L // _NS
_NB = 512
_NW = 32
_RW = _N // _NW


def _perm_matrix(L, ns):
    c = L // ns
    src = jnp.arange(L)
    dst = (src % ns) * c + src // ns
    return jnp.zeros((L, L), jnp.float32).at[src, dst].set(1.0)


def _pre_body(e_ref, p_ref, ed_ref):
    ed_ref[...] = jnp.dot(e_ref[...], p_ref[...],
                          preferred_element_type=jnp.float32)


def _tc_body(x_ref, ed_ref, p_ref, o1_ref):
    xb = x_ref[0]
    ed = ed_ref[...]
    pm = p_ref[...]
    xd = jnp.dot(xb, pm, preferred_element_type=jnp.float32)
    for s in range(_NS):
        o1_ref[0, :, s, :] = jnp.concatenate(
            [xd[:, s * _C:(s + 1) * _C], ed[:, s * _C:(s + 1) * _C]], axis=1)


def _tc_out1(x, node_emb):
    B, N, L = x.shape
    pm = _perm_matrix(L, _NS)
    ed_all = pl.pallas_call(
        _pre_body,
        grid=(N // _NB,),
        in_specs=[
            pl.BlockSpec((_NB, L), lambda i: (i, 0)),
            pl.BlockSpec((L, L), lambda i: (0, 0)),
        ],
        out_specs=pl.BlockSpec((_NB, L), lambda i: (i, 0)),
        out_shape=jax.ShapeDtypeStruct((N, L), jnp.float32),
    )(node_emb, pm)

    out4 = jax.ShapeDtypeStruct((B, N, _NS, 2 * _C), x.dtype)
    return pl.pallas_call(
        _tc_body,
        grid=(N // _NB, B),
        in_specs=[
            pl.BlockSpec((1, _NB, L), lambda i, b: (b, i, 0)),
            pl.BlockSpec((_NB, L), lambda i, b: (i, 0)),
            pl.BlockSpec((L, L), lambda i, b: (0, 0)),
        ],
        out_specs=pl.BlockSpec((1, _NB, _NS, 2 * _C),
                               lambda i, b: (b, i, 0, 0)),
        out_shape=out4,
    )(x, ed_all, pm)


def _sc_out2(x, node_emb):
    mesh = plsc.VectorSubcoreMesh(core_axis_name="c", subcore_axis_name="s")
    out_sd = jax.ShapeDtypeStruct((_B, _N, _NS, 2 * _C), jnp.float32)

    @functools.partial(
        pl.kernel, mesh=mesh,
        out_type=out_sd,
        scratch_types=[
            pltpu.VMEM((2, _RW, _L), jnp.float32),
            pltpu.VMEM((2, _RW, _NS, 2 * _C), jnp.float32),
            pltpu.SemaphoreType.DMA((2,)),
            pltpu.SemaphoreType.DMA((2,)),
        ],
    )
    def k(x_hbm, e_hbm, o2_hbm, xv, s2, sin, sout):
        wid = lax.axis_index("s") * 2 + lax.axis_index("c")
        n0 = wid * _RW

        def in_copy(b, buf):
            return pltpu.make_async_copy(
                x_hbm.at[b, pl.ds(n0, _RW)], xv.at[buf], sin.at[buf])

        def out_copy(b, buf):
            return pltpu.make_async_copy(
                s2.at[buf], o2_hbm.at[b, pl.ds(n0, _RW)], sout.at[buf])

        def fill(buf, r, xoff):
            for s in range(_NS):
                for g in range(_C // 16):
                    v = xv[buf, r, pl.ds(s * _C + g * 16, 16)]
                    s2[buf, r, s, pl.ds(xoff + g * 16, 16)] = v

        pltpu.sync_copy(e_hbm.at[pl.ds(n0, _RW)], xv.at[0])

        def fill_emb(r, carry):
            fill(0, r, _C)
            return carry

        lax.fori_loop(0, _RW, fill_emb, 0)

        def fill_emb1(r, carry):
            for s in range(_NS):
                for g in range(_C // 16):
                    v = s2[0, r, s, pl.ds(_C + g * 16, 16)]
                    s2[1, r, s, pl.ds(_C + g * 16, 16)] = v
            return carry

        lax.fori_loop(0, _RW, fill_emb1, 0)

        in_copy(0, 0).start()
        for b in range(_B):
            buf = b % 2
            in_copy(b, buf).wait()
            if b + 1 < _B:
                in_copy(b + 1, 1 - buf).start()
            if b >= 2:
                out_copy(b - 2, buf).wait()

            def fill_x(r, c2, buf=buf):
                fill(buf, r, 0)
                return c2

            lax.fori_loop(0, _RW, fill_x, 0)
            out_copy(b, buf).start()
        out_copy(_B - 2, 0).wait()
        out_copy(_B - 1, 1).wait()

    return k(x, node_emb)


def kernel(x, x_mark, node_emb):
    del x_mark
    o1 = _tc_out1(x, node_emb)
    o2 = _sc_out2(x, node_emb)
    return o1, o2


# speedup vs baseline: 1.2525x; 1.0468x over previous
import functools
import jax
import jax.numpy as jnp
from jax import lax
from jax.experimental import pallas as pl
from jax.experimental.pallas import tpu as pltpu
from jax.experimental.pallas import tpu_sc as plsc

_B, _N, _L = 16, 1024, 512
_NS = 4
_C = _L // _NS
_NB = 1024
_NW = 32
_RW = _N // _NW


def _perm_matrix(L, ns):
    c = L // ns
    src = jnp.arange(L)
    dst = (src % ns) * c + src // ns
    return jnp.zeros((L, L), jnp.float32).at[src, dst].set(1.0)


def _pre_body(e_ref, p_ref, ed_ref):
    ed_ref[...] = jnp.dot(e_ref[...], p_ref[...],
                          preferred_element_type=jnp.float32)


def _tc_body(x_ref, ed_ref, p_ref, o1_ref):
    xb = x_ref[0]
    ed = ed_ref[...]
    pm = p_ref[...]
    xd = jnp.dot(xb, pm, preferred_element_type=jnp.float32)
    for s in range(_NS):
        o1_ref[0, :, s, :] = jnp.concatenate(
            [xd[:, s * _C:(s + 1) * _C], ed[:, s * _C:(s + 1) * _C]], axis=1)


def _tc_out1(x, node_emb):
    B, N, L = x.shape
    pm = _perm_matrix(L, _NS)
    ed_all = pl.pallas_call(
        _pre_body,
        grid=(N // _NB,),
        in_specs=[
            pl.BlockSpec((_NB, L), lambda i: (i, 0)),
            pl.BlockSpec((L, L), lambda i: (0, 0)),
        ],
        out_specs=pl.BlockSpec((_NB, L), lambda i: (i, 0)),
        out_shape=jax.ShapeDtypeStruct((N, L), jnp.float32),
    )(node_emb, pm)

    out4 = jax.ShapeDtypeStruct((B, N, _NS, 2 * _C), x.dtype)
    return pl.pallas_call(
        _tc_body,
        grid=(N // _NB, B),
        in_specs=[
            pl.BlockSpec((1, _NB, L), lambda i, b: (b, i, 0)),
            pl.BlockSpec((_NB, L), lambda i, b: (i, 0)),
            pl.BlockSpec((L, L), lambda i, b: (0, 0)),
        ],
        out_specs=pl.BlockSpec((1, _NB, _NS, 2 * _C),
                               lambda i, b: (b, i, 0, 0)),
        out_shape=out4,
    )(x, ed_all, pm)


def _sc_out2(x, node_emb):
    mesh = plsc.VectorSubcoreMesh(core_axis_name="c", subcore_axis_name="s")
    out_sd = jax.ShapeDtypeStruct((_B, _N, _NS, 2 * _C), jnp.float32)

    @functools.partial(
        pl.kernel, mesh=mesh,
        out_type=out_sd,
        scratch_types=[
            pltpu.VMEM((2, _RW, _L), jnp.float32),
            pltpu.VMEM((2, _RW, _NS, 2 * _C), jnp.float32),
            pltpu.SemaphoreType.DMA((2,)),
            pltpu.SemaphoreType.DMA((2,)),
        ],
    )
    def k(x_hbm, e_hbm, o2_hbm, xv, s2, sin, sout):
        wid = lax.axis_index("s") * 2 + lax.axis_index("c")
        n0 = wid * _RW

        def in_copy(b, buf):
            return pltpu.make_async_copy(
                x_hbm.at[b, pl.ds(n0, _RW)], xv.at[buf], sin.at[buf])

        def out_copy(b, buf):
            return pltpu.make_async_copy(
                s2.at[buf], o2_hbm.at[b, pl.ds(n0, _RW)], sout.at[buf])

        def fill(buf, r, xoff):
            for s in range(_NS):
                for g in range(_C // 16):
                    v = xv[buf, r, pl.ds(s * _C + g * 16, 16)]
                    s2[buf, r, s, pl.ds(xoff + g * 16, 16)] = v

        pltpu.sync_copy(e_hbm.at[pl.ds(n0, _RW)], xv.at[0])

        def fill_emb(r, carry):
            fill(0, r, _C)
            return carry

        lax.fori_loop(0, _RW, fill_emb, 0)

        def fill_emb1(r, carry):
            for s in range(_NS):
                for g in range(_C // 16):
                    v = s2[0, r, s, pl.ds(_C + g * 16, 16)]
                    s2[1, r, s, pl.ds(_C + g * 16, 16)] = v
            return carry

        lax.fori_loop(0, _RW, fill_emb1, 0)

        in_copy(0, 0).start()
        for b in range(_B):
            buf = b % 2
            in_copy(b, buf).wait()
            if b + 1 < _B:
                in_copy(b + 1, 1 - buf).start()
            if b >= 2:
                out_copy(b - 2, buf).wait()

            def fill_x(r, c2, buf=buf):
                fill(buf, r, 0)
                return c2

            lax.fori_loop(0, _RW, fill_x, 0)
            out_copy(b, buf).start()
        out_copy(_B - 2, 0).wait()
        out_copy(_B - 1, 1).wait()

    return k(x, node_emb)


def kernel(x, x_mark, node_emb):
    del x_mark
    o1 = _tc_out1(x, node_emb)
    o2 = _sc_out2(x, node_emb)
    return o1, o2
